# Initial kernel scaffold; baseline (speedup 1.0000x reference)
import functools

import jax
import jax.numpy as jnp
from jax import lax
from jax.experimental import pallas as pl
from jax.experimental.pallas import tpu as pltpu
from jax.experimental.pallas import tpu_sc as plsc

_B = 16384
_L = 50
_D = 32
_NC = 2
_NS = 16
_NW = _NC * _NS
_BPW = _B // _NW
_CB = 64
_NCHUNK = _BPW // _CB
_IDXROWS = _CB * _L // 128
_CLICK_ROWS_PER_W = _BPW * _L // 128

_mesh = plsc.VectorSubcoreMesh(core_axis_name="c", subcore_axis_name="s")


@functools.partial(
    pl.kernel,
    out_type=jax.ShapeDtypeStruct((_B, 4 * _D), jnp.float32),
    mesh=_mesh,
    scratch_types=[
        pltpu.VMEM((_IDXROWS, 128), jnp.int32),
        pltpu.VMEM((_CB * _L, _D), jnp.float32),
        pltpu.VMEM((_CB,), jnp.int32),
        pltpu.VMEM((_CB,), jnp.int32),
        pltpu.VMEM((_CB,), jnp.int32),
        pltpu.VMEM((_CB, _D), jnp.float32),
        pltpu.VMEM((_CB, _D), jnp.float32),
        pltpu.VMEM((_CB, _D), jnp.float32),
        pltpu.VMEM((_CB, 4 * _D), jnp.float32),
        pltpu.SemaphoreType.DMA,
    ],
)
def _sc_embed(uid_hbm, age_hbm, gen_hbm, clicks_hbm, ut_hbm, at_hbm, gt_hbm,
              it_hbm, out_hbm, idx_v, rows_v, uidx_v, aidx_v, gidx_v,
              urows_v, arows_v, grows_v, outc_v, sem):
    wid = lax.axis_index("s") * _NC + lax.axis_index("c")
    base = wid * _BPW
    cbase = wid * _CLICK_ROWS_PER_W
    inv = jnp.full((16,), 1.0 / _L, dtype=jnp.float32)

    def chunk_body(c, carry):
        rowbase = pl.multiple_of(base + c * _CB, _CB)
        ibase = cbase + c * _IDXROWS
        pltpu.sync_copy(clicks_hbm.at[pl.ds(ibase, _IDXROWS)], idx_v)
        copies = []
        for j in range(_IDXROWS):
            copies.append(pltpu.async_copy(
                it_hbm.at[idx_v.at[j]], rows_v.at[pl.ds(j * 128, 128)], sem))
        pltpu.sync_copy(uid_hbm.at[pl.ds(rowbase, _CB)], uidx_v)
        pltpu.sync_copy(age_hbm.at[pl.ds(rowbase, _CB)], aidx_v)
        pltpu.sync_copy(gen_hbm.at[pl.ds(rowbase, _CB)], gidx_v)
        cu = pltpu.async_copy(ut_hbm.at[uidx_v], urows_v, sem)
        ca = pltpu.async_copy(at_hbm.at[aidx_v], arows_v, sem)
        cg = pltpu.async_copy(gt_hbm.at[gidx_v], grows_v, sem)
        for cp in copies:
            cp.wait()
        cu.wait()
        ca.wait()
        cg.wait()

        def row_body(r, rcarry):
            rb = r * _L
            a0 = jnp.zeros((16,), jnp.float32)
            a1 = jnp.zeros((16,), jnp.float32)
            for jj in range(_L):
                a0 = a0 + rows_v[rb + jj, pl.ds(0, 16)]
                a1 = a1 + rows_v[rb + jj, pl.ds(16, 16)]
            outc_v[r, pl.ds(0, 16)] = urows_v[r, pl.ds(0, 16)]
            outc_v[r, pl.ds(16, 16)] = urows_v[r, pl.ds(16, 16)]
            outc_v[r, pl.ds(32, 16)] = arows_v[r, pl.ds(0, 16)]
            outc_v[r, pl.ds(48, 16)] = arows_v[r, pl.ds(16, 16)]
            outc_v[r, pl.ds(64, 16)] = grows_v[r, pl.ds(0, 16)]
            outc_v[r, pl.ds(80, 16)] = grows_v[r, pl.ds(16, 16)]
            outc_v[r, pl.ds(96, 16)] = a0 * inv
            outc_v[r, pl.ds(112, 16)] = a1 * inv
            return rcarry

        lax.fori_loop(0, _CB, row_body, 0)
        pltpu.sync_copy(outc_v, out_hbm.at[pl.ds(rowbase, _CB)])
        return carry

    lax.fori_loop(0, _NCHUNK, chunk_body, 0)


def _dense_body(x_ref, w_ref, b_ref, o_ref):
    acc = jnp.dot(x_ref[...], w_ref[...], preferred_element_type=jnp.float32)
    o_ref[...] = jnp.maximum(acc + b_ref[...], 0.0)


_BM = 1024
_dense = pl.pallas_call(
    _dense_body,
    grid=(_B // _BM,),
    in_specs=[
        pl.BlockSpec((_BM, 4 * _D), lambda i: (i, 0)),
        pl.BlockSpec((4 * _D, 64), lambda i: (0, 0)),
        pl.BlockSpec((1, 64), lambda i: (0, 0)),
    ],
    out_specs=pl.BlockSpec((_BM, 64), lambda i: (i, 0)),
    out_shape=jax.ShapeDtypeStruct((_B, 64), jnp.float32),
)


def kernel(user_id, age, gender, recent_clicks, user_table, age_table,
           gender_table, item_table, W, b):
    clicks2d = recent_clicks.reshape(_B * _L // 128, 128)
    concat = _sc_embed(user_id, age, gender, clicks2d, user_table, age_table,
                       gender_table, item_table)
    return _dense(concat, W, b.reshape(1, 64))


# speedup vs baseline: 2.1832x; 2.1832x over previous
import functools

import jax
import jax.numpy as jnp
from jax import lax
from jax.experimental import pallas as pl
from jax.experimental.pallas import tpu as pltpu
from jax.experimental.pallas import tpu_sc as plsc

_B = 16384
_L = 50
_D = 32
_NC = 2
_NS = 16
_NW = _NC * _NS
_BPW = _B // _NW
_CB = 64
_NCHUNK = _BPW // _CB
_NGATHER = _CB * _L // 128

_mesh = plsc.VectorSubcoreMesh(core_axis_name="c", subcore_axis_name="s")


@functools.partial(
    pl.kernel,
    out_type=jax.ShapeDtypeStruct((_B, 4 * _D), jnp.float32),
    mesh=_mesh,
    scratch_types=[
        pltpu.VMEM((_CB * _L,), jnp.int32),
        pltpu.VMEM((_CB * _L, _D), jnp.float32),
        pltpu.VMEM((_CB,), jnp.int32),
        pltpu.VMEM((_CB,), jnp.int32),
        pltpu.VMEM((_CB,), jnp.int32),
        pltpu.VMEM((_CB, _D), jnp.float32),
        pltpu.VMEM((_CB, _D), jnp.float32),
        pltpu.VMEM((_CB, _D), jnp.float32),
        pltpu.VMEM((_CB, 4 * _D), jnp.float32),
        pltpu.SemaphoreType.DMA,
    ],
    compiler_params=pltpu.CompilerParams(use_tc_tiling_on_sc=False),
)
def _sc_embed(uid_hbm, age_hbm, gen_hbm, clicks_hbm, ut_hbm, at_hbm, gt_hbm,
              it_hbm, out_hbm, idx_v, rows_v, uidx_v, aidx_v, gidx_v,
              urows_v, arows_v, grows_v, outc_v, sem):
    wid = lax.axis_index("s") * _NC + lax.axis_index("c")
    base = wid * _BPW
    inv = jnp.full((16,), 1.0 / _L, dtype=jnp.float32)

    def chunk_body(c, carry):
        rowbase = pl.multiple_of(base + c * _CB, _CB)
        ibase = pl.multiple_of(rowbase * _L, _CB * _L)
        pltpu.sync_copy(clicks_hbm.at[pl.ds(ibase, _CB * _L)], idx_v)
        copies = []
        for j in range(_NGATHER):
            copies.append(pltpu.async_copy(
                it_hbm.at[idx_v.at[pl.ds(j * 128, 128)]],
                rows_v.at[pl.ds(j * 128, 128)], sem))
        pltpu.sync_copy(uid_hbm.at[pl.ds(rowbase, _CB)], uidx_v)
        pltpu.sync_copy(age_hbm.at[pl.ds(rowbase, _CB)], aidx_v)
        pltpu.sync_copy(gen_hbm.at[pl.ds(rowbase, _CB)], gidx_v)
        cu = pltpu.async_copy(ut_hbm.at[uidx_v], urows_v, sem)
        ca = pltpu.async_copy(at_hbm.at[aidx_v], arows_v, sem)
        cg = pltpu.async_copy(gt_hbm.at[gidx_v], grows_v, sem)
        for cp in copies:
            cp.wait()
        cu.wait()
        ca.wait()
        cg.wait()

        def row_body(r, rcarry):
            rb = r * _L
            a0 = jnp.zeros((16,), jnp.float32)
            a1 = jnp.zeros((16,), jnp.float32)
            for jj in range(_L):
                a0 = a0 + rows_v[rb + jj, pl.ds(0, 16)]
                a1 = a1 + rows_v[rb + jj, pl.ds(16, 16)]
            outc_v[r, pl.ds(0, 16)] = urows_v[r, pl.ds(0, 16)]
            outc_v[r, pl.ds(16, 16)] = urows_v[r, pl.ds(16, 16)]
            outc_v[r, pl.ds(32, 16)] = arows_v[r, pl.ds(0, 16)]
            outc_v[r, pl.ds(48, 16)] = arows_v[r, pl.ds(16, 16)]
            outc_v[r, pl.ds(64, 16)] = grows_v[r, pl.ds(0, 16)]
            outc_v[r, pl.ds(80, 16)] = grows_v[r, pl.ds(16, 16)]
            outc_v[r, pl.ds(96, 16)] = a0 * inv
            outc_v[r, pl.ds(112, 16)] = a1 * inv
            return rcarry

        lax.fori_loop(0, _CB, row_body, 0)
        pltpu.sync_copy(outc_v, out_hbm.at[pl.ds(rowbase, _CB)])
        return carry

    lax.fori_loop(0, _NCHUNK, chunk_body, 0)


def _dense_body(x_ref, w_ref, b_ref, o_ref):
    acc = jnp.dot(x_ref[...], w_ref[...], preferred_element_type=jnp.float32)
    o_ref[...] = jnp.maximum(acc + b_ref[...], 0.0)


_BM = 1024
_dense = pl.pallas_call(
    _dense_body,
    grid=(_B // _BM,),
    in_specs=[
        pl.BlockSpec((_BM, 4 * _D), lambda i: (i, 0)),
        pl.BlockSpec((4 * _D, 64), lambda i: (0, 0)),
        pl.BlockSpec((1, 64), lambda i: (0, 0)),
    ],
    out_specs=pl.BlockSpec((_BM, 64), lambda i: (i, 0)),
    out_shape=jax.ShapeDtypeStruct((_B, 64), jnp.float32),
)


def kernel(user_id, age, gender, recent_clicks, user_table, age_table,
           gender_table, item_table, W, b):
    clicks_flat = recent_clicks.reshape(_B * _L)
    concat = _sc_embed(user_id, age, gender, clicks_flat, user_table,
                       age_table, gender_table, item_table)
    return _dense(concat, W, b.reshape(1, 64))
